# baseline (speedup 1.0000x reference)
import functools

import jax
import jax.numpy as jnp
from jax import lax
from jax.experimental import pallas as pl
from jax.experimental.pallas import tpu as pltpu
from jax.experimental.pallas import tpu_sc as plsc


@functools.lru_cache(maxsize=None)
def _build_gather(B2, A, V, D):
    info = plsc.get_sparse_core_info()
    NC, NS = info.num_cores, info.num_subcores
    NW = NC * NS
    L = 128
    DG = D // 8
    JT = B2 // L
    j_per_w = JT // NW
    n_units = A * j_per_w
    assert JT % NW == 0 and D % 8 == 0 and n_units % 2 == 0

    mesh = plsc.VectorSubcoreMesh(core_axis_name="c", subcore_axis_name="s")

    @functools.partial(
        pl.kernel,
        mesh=mesh,
        compiler_params=pltpu.CompilerParams(
            use_tc_tiling_on_sc=False, needs_layout_passes=False),
        out_type=jax.ShapeDtypeStruct((A, DG, JT, 8, L), jnp.float32),
        scratch_types=[
            pltpu.VMEM((A * j_per_w * L,), jnp.int32),
            pltpu.VMEM((L, D), jnp.float32),
            pltpu.VMEM((L, D), jnp.float32),
            pltpu.VMEM((D, L), jnp.float32),
            pltpu.VMEM((D, L), jnp.float32),
            pltpu.SemaphoreType.DMA,
            pltpu.SemaphoreType.DMA,
            pltpu.SemaphoreType.DMA,
            pltpu.SemaphoreType.DMA,
            pltpu.SemaphoreType.DMA,
        ],
    )
    def k(xt_hbm, table_hbm, o5_hbm, idx_v, r0, r1, st0, st1,
          isem, g0, g1, s0, s1):
        wid = lax.axis_index("s") * NC + lax.axis_index("c")
        jbase = wid * j_per_w
        rows = (r0, r1)
        sts = (st0, st1)
        gsems = (g0, g1)
        ssems = (s0, s1)

        span = j_per_w * L

        def stage_idx(a, _):
            pltpu.async_copy(
                xt_hbm.at[pl.ds(a * B2 + jbase * L, span)],
                idx_v.at[pl.ds(a * span, span)], isem)
            return 0

        def drain_idx(a, _):
            pltpu.make_async_copy(
                xt_hbm.at[pl.ds(0, span)],
                idx_v.at[pl.ds(0, span)], isem).wait()
            return 0

        lax.fori_loop(0, A, stage_idx, 0, unroll=False)
        lax.fori_loop(0, A, drain_idx, 0, unroll=False)

        def gather(u, b):
            pltpu.async_copy(
                table_hbm.at[idx_v.at[pl.ds(u * L, L)]], rows[b], gsems[b])

        def wait_gather(b):
            pltpu.make_async_copy(
                table_hbm.at[pl.ds(0, L)], rows[b], gsems[b]).wait()

        def wait_store(b):
            for i in range(DG):
                pltpu.make_async_copy(
                    sts[b].at[pl.ds(8 * i, 8)],
                    o5_hbm.at[0, i, jbase], ssems[b]).wait()

        iota = lax.iota(jnp.int32, 16)
        rpat = [(iota + t) & 15 for t in range(16)]

        def do_unit(p, h):
            u = 2 * p + h
            a = u // j_per_w
            j = jbase + (u % j_per_w)
            @pl.when(p >= 1)
            def _():
                wait_store(h)
            wait_gather(h)
            if h == 0:
                gather(u + 1, 1)
            else:
                @pl.when(p < n_units // 2 - 1)
                def _():
                    gather(u + 1, 0)
            def tblock(lb, _):
                lvec = iota + lb * 16
                for t in range(16):
                    for c0 in range(0, D, 16):
                        cvec = rpat[t] + c0 if c0 else rpat[t]
                        v = plsc.load_gather(rows[h], [lvec, cvec])
                        plsc.store_scatter(sts[h], [cvec, lvec], v)
                return 0

            lax.fori_loop(0, L // 16, tblock, 0, unroll=2)
            for i in range(DG):
                pltpu.async_copy(
                    sts[h].at[pl.ds(8 * i, 8)], o5_hbm.at[a, i, j], ssems[h])

        gather(0, 0)

        def pair(p, _):
            do_unit(p, 0)
            do_unit(p, 1)
            return 0

        lax.fori_loop(0, n_units // 2, pair, 0, unroll=False)
        wait_store(0)
        wait_store(1)

    return k


def kernel(x, table):
    V, D = table.shape
    B2, A = x.shape
    xt = (jnp.transpose(x).reshape(-1) * 2).astype(jnp.int32)
    table2 = jnp.repeat(table, 2, axis=0)
    o5 = _build_gather(B2, A, 2 * V, D)(xt, table2)
    out = o5.transpose(2, 4, 0, 1, 3).reshape(B2, A, D)
    return out


# speedup vs baseline: 2.2323x; 2.2323x over previous
import functools

import jax
import jax.numpy as jnp
from jax import lax
from jax.experimental import pallas as pl
from jax.experimental.pallas import tpu as pltpu
from jax.experimental.pallas import tpu_sc as plsc


@functools.lru_cache(maxsize=None)
def _build_gather(B2, A, V, D):
    info = plsc.get_sparse_core_info()
    NC, NS = info.num_cores, info.num_subcores
    NW = NC * NS
    L = 128
    DG = D // 8
    JT = B2 // L
    j_per_w = JT // NW
    n_units = A * j_per_w
    assert JT % NW == 0 and D % 8 == 0 and n_units % 2 == 0

    mesh = plsc.VectorSubcoreMesh(core_axis_name="c", subcore_axis_name="s")

    @functools.partial(
        pl.kernel,
        mesh=mesh,
        compiler_params=pltpu.CompilerParams(
            use_tc_tiling_on_sc=False, needs_layout_passes=False),
        out_type=jax.ShapeDtypeStruct((A, DG, JT, 8, L), jnp.float32),
        scratch_types=[
            pltpu.VMEM((A * j_per_w * L,), jnp.int32),
            pltpu.VMEM((L, D), jnp.float32),
            pltpu.VMEM((L, D), jnp.float32),
            pltpu.VMEM((D, L), jnp.float32),
            pltpu.VMEM((D, L), jnp.float32),
            pltpu.SemaphoreType.DMA,
            pltpu.SemaphoreType.DMA,
            pltpu.SemaphoreType.DMA,
            pltpu.SemaphoreType.DMA,
            pltpu.SemaphoreType.DMA,
        ],
    )
    def k(xt_hbm, table_hbm, o5_hbm, idx_v, r0, r1, st0, st1,
          isem, g0, g1, s0, s1):
        wid = lax.axis_index("s") * NC + lax.axis_index("c")
        jbase = wid * j_per_w
        rows = (r0, r1)
        sts = (st0, st1)
        gsems = (g0, g1)
        ssems = (s0, s1)

        span = j_per_w * L

        def stage_idx(a, _):
            pltpu.async_copy(
                xt_hbm.at[pl.ds(a * B2 + jbase * L, span)],
                idx_v.at[pl.ds(a * span, span)], isem)
            return 0

        def drain_idx(a, _):
            pltpu.make_async_copy(
                xt_hbm.at[pl.ds(0, span)],
                idx_v.at[pl.ds(0, span)], isem).wait()
            return 0

        lax.fori_loop(0, A, stage_idx, 0, unroll=False)
        lax.fori_loop(0, A, drain_idx, 0, unroll=False)

        def gather(u, b):
            pltpu.async_copy(
                table_hbm.at[idx_v.at[pl.ds(u * L, L)]], rows[b], gsems[b])

        def wait_gather(b):
            pltpu.make_async_copy(
                table_hbm.at[pl.ds(0, L)], rows[b], gsems[b]).wait()

        def wait_store(b):
            for i in range(DG):
                pltpu.make_async_copy(
                    sts[b].at[pl.ds(8 * i, 8)],
                    o5_hbm.at[0, i, jbase], ssems[b]).wait()

        iota = lax.iota(jnp.int32, 16)
        rpat = [(iota + t) & 15 for t in range(16)]

        def do_unit(p, h):
            u = 2 * p + h
            a = u // j_per_w
            j = jbase + (u % j_per_w)
            @pl.when(p >= 1)
            def _():
                wait_store(h)
            wait_gather(h)
            if h == 0:
                gather(u + 1, 1)
            else:
                @pl.when(p < n_units // 2 - 1)
                def _():
                    gather(u + 1, 0)
            def tblock(lb, _):
                lvec = iota + lb * 16
                for t in range(16):
                    for c0 in range(0, D, 16):
                        cvec = rpat[t] + c0 if c0 else rpat[t]
                        v = plsc.load_gather(rows[h], [lvec, cvec])
                        plsc.store_scatter(sts[h], [cvec, lvec], v)
                return 0

            lax.fori_loop(0, L // 16, tblock, 0, unroll=2)
            for i in range(DG):
                pltpu.async_copy(
                    sts[h].at[pl.ds(8 * i, 8)], o5_hbm.at[a, i, j], ssems[h])

        gather(0, 0)

        def pair(p, _):
            do_unit(p, 0)
            do_unit(p, 1)
            return 0

        lax.fori_loop(0, n_units // 2, pair, 0, unroll=False)
        wait_store(0)
        wait_store(1)

    return k


def kernel(x, table):
    V, D = table.shape
    B2, A = x.shape
    xt = (jnp.transpose(x).reshape(-1) * 4).astype(jnp.int32)
    table4 = jnp.pad(table, ((0, 0), (0, 3 * D))).reshape(4 * V, D)
    o5 = _build_gather(B2, A, 4 * V, D)(xt, table4)
    out = o5.transpose(2, 4, 0, 1, 3).reshape(B2, A, D)
    return out
